# baseline (speedup 1.0000x reference)
import jax
import jax.numpy as jnp
from jax.experimental import pallas as pl
from jax.experimental.pallas import tpu as pltpu

N = 512
BI = 128
STACK = 32
F32 = jnp.float32


def _edge_step(idx, x_ref, edge_ref, a_ref, f, ws_ref, bs_ref,
               wai_ref, wao_ref, we_out_ref, bvec,
               e1_scr, min_scr, mout_scr, vt_scr, emit_edge):
    rows = pl.ds(idx * BI, BI)
    wsx = ws_ref[0:f, :]
    we = ws_ref[2 * f:2 * f + 1, :].reshape(1, STACK, 1)
    wet = ws_ref[2 * f + 1:2 * f + 2, :].reshape(1, STACK, 1)
    u = jnp.dot(x_ref[rows, :], wsx, preferred_element_type=F32) + bs_ref[...]
    vt = vt_scr[...]
    e_blk = edge_ref[rows, :]
    et_blk = edge_ref[:, rows].T
    g = jnp.concatenate(
        [jnp.broadcast_to(we.reshape(1, STACK, 1), (BI, STACK, 1)),
         jnp.broadcast_to(wet.reshape(1, STACK, 1), (BI, STACK, 1))],
        axis=2)
    h = jnp.concatenate(
        [e_blk[:, None, :], et_blk[:, None, :]], axis=1)
    z = jax.lax.dot_general(g, h, (((2,), (1,)), ((0,), (0,))),
                            preferred_element_type=F32)
    t = jnp.maximum(z + u[:, :, None] + vt[None, :, :], 0.0)
    t = t.astype(jnp.bfloat16)

    heads = [wai_ref[...], wao_ref[...]]
    if emit_edge:
        heads.append(we_out_ref[...])
    w3 = jnp.concatenate(heads, axis=1).T.astype(jnp.bfloat16)
    k = len(heads)
    w3b = jnp.broadcast_to(w3[None, :, :], (BI, k, STACK))
    proj = jax.lax.dot_general(w3b, t, (((2,), (1,)), ((0,), (0,))),
                               preferred_element_type=F32)
    att_i = jax.nn.sigmoid(proj[:, 0, :] + bvec[0])
    att_j = jax.nn.sigmoid(proj[:, 1, :] + bvec[1])
    mask = (a_ref[rows, :] != 0.0).astype(F32)
    if emit_edge:
        e1_scr[rows, :] = proj[:, 2, :] + bvec[2]
    w_in = mask * att_i
    w_out = mask * att_j
    min_scr[rows, :] = jax.lax.dot_general(
        t, w_in.astype(jnp.bfloat16), (((2,), (1,)), ((0,), (0,))),
        preferred_element_type=F32)
    mout_scr[...] += jnp.sum((t * w_out[:, None, :].astype(jnp.bfloat16)
                              ).astype(F32), axis=0)


def _node_update(x, min_scr, mout_scr, f, wn_ref, bn_ref):
    xn = jnp.dot(x, wn_ref[0:f, :], preferred_element_type=F32)
    xn += jnp.dot(min_scr[...], wn_ref[f:f + STACK, :],
                  preferred_element_type=F32)
    xn += jax.lax.dot_general(mout_scr[...], wn_ref[f + STACK:, :],
                              (((0,), (0,)), ((), ())),
                              preferred_element_type=F32)
    return xn + bn_ref[...]


def _make_fused(f1, node):
    def _fused_kernel(x_ref, a_ref, e_ref,
                      ws1_ref, bs1_ref, wai1_ref, bai1_ref, wao1_ref,
                      bao1_ref, wn1_ref, bn1_ref, we1_ref, be1_ref,
                      ws2_ref, bs2_ref, wai2_ref, bai2_ref, wao2_ref,
                      bao2_ref, wn2_ref, bn2_ref, we2_ref,
                      wd_ref, bd_ref, out_ref,
                      e1_scr, min_scr, mout_scr, x1_scr, vt_scr):
        s = pl.program_id(0)

        @pl.when(s == 0)
        def _zero():
            mout_scr[...] = jnp.zeros_like(mout_scr)
            vt_scr[...] = jax.lax.dot_general(
                ws1_ref[f1:2 * f1, :], x_ref[...], (((0,), (1,)), ((), ())),
                preferred_element_type=F32)

        @pl.when(s < 2)
        def _layer1():
            for half in range(2):
                _edge_step(s * 2 + half, x_ref, e_ref, a_ref, f1, ws1_ref,
                           bs1_ref, wai1_ref, wao1_ref, we1_ref,
                           (bai1_ref[0, 0], bao1_ref[0, 0], be1_ref[0, 0]),
                           e1_scr, min_scr, mout_scr, vt_scr, emit_edge=True)

        @pl.when(s == 2)
        def _node1():
            x1 = _node_update(x_ref[...], min_scr, mout_scr, f1,
                              wn1_ref, bn1_ref)
            x1_scr[...] = x1
            mout_scr[...] = jnp.zeros_like(mout_scr)
            vt_scr[...] = jax.lax.dot_general(
                ws2_ref[node:2 * node, :], x1, (((0,), (1,)), ((), ())),
                preferred_element_type=F32)

        @pl.when(jnp.logical_and(s >= 3, s < 5))
        def _layer2():
            for half in range(2):
                _edge_step((s - 3) * 2 + half, x1_scr, e1_scr, a_ref, node,
                           ws2_ref, bs2_ref, wai2_ref, wao2_ref, we2_ref,
                           (bai2_ref[0, 0], bao2_ref[0, 0], 0.0),
                           None, min_scr, mout_scr, vt_scr, emit_edge=False)

        @pl.when(s == 5)
        def _node2():
            x2 = _node_update(x1_scr[...], min_scr, mout_scr, node,
                              wn2_ref, bn2_ref)
            out_ref[...] = jnp.dot(x2, wd_ref[...],
                                   preferred_element_type=F32) + bd_ref[...]

    return _fused_kernel


def kernel(x, a, e, Ws1, bs1, Wai1, bai1, Wao1, bao1, Wn1, bn1, We1, be1,
           Ws2, bs2, Wai2, bai2, Wao2, bao2, Wn2, bn2, We2, be2, Wd, bd):
    f1 = x.shape[2]
    node = Wn1.shape[1]
    nlab = Wd.shape[1]

    operands = [
        x[0], a[0], e[0, :, :, 0],
        Ws1, bs1.reshape(1, STACK), Wai1, bai1.reshape(1, 1), Wao1,
        bao1.reshape(1, 1), Wn1, bn1.reshape(1, node), We1, be1.reshape(1, 1),
        Ws2, bs2.reshape(1, STACK), Wai2, bai2.reshape(1, 1), Wao2,
        bao2.reshape(1, 1), Wn2, bn2.reshape(1, node), We2,
        Wd, bd.reshape(1, nlab),
    ]
    full = lambda s: pl.BlockSpec(s, lambda i: (0,) * len(s))
    out = pl.pallas_call(
        _make_fused(f1, node),
        grid=(6,),
        in_specs=[full(op.shape) for op in operands],
        out_shape=jax.ShapeDtypeStruct((N, nlab), F32),
        out_specs=full((N, nlab)),
        scratch_shapes=[
            pltpu.VMEM((N, N), F32),
            pltpu.VMEM((N, STACK), F32),
            pltpu.VMEM((STACK, N), F32),
            pltpu.VMEM((N, node), F32),
            pltpu.VMEM((STACK, N), F32),
        ],
    )(*operands)
    return out[None, :, :]


# speedup vs baseline: 1.0386x; 1.0386x over previous
import jax
import jax.numpy as jnp
from jax.experimental import pallas as pl
from jax.experimental.pallas import tpu as pltpu

N = 512
BI = 128
STACK = 32
F32 = jnp.float32


def _edge_step(idx, x_ref, edge_ref, a_ref, f, ws_ref, bs_ref,
               wai_ref, wao_ref, we_out_ref, bvec,
               e1_scr, min_scr, mout_scr, emit_edge):
    rows = pl.ds(idx * BI, BI)
    wsx = ws_ref[0:f, :]
    wsv = ws_ref[f:2 * f, :]
    we = ws_ref[2 * f:2 * f + 1, :].reshape(1, STACK, 1)
    wet = ws_ref[2 * f + 1:2 * f + 2, :].reshape(1, STACK, 1)
    u = jnp.dot(x_ref[rows, :], wsx, preferred_element_type=F32) + bs_ref[...]
    vt = jax.lax.dot_general(wsv, x_ref[...], (((0,), (1,)), ((), ())),
                             preferred_element_type=F32)
    e_blk = edge_ref[rows, :]
    et_blk = edge_ref[:, rows].T
    g = jnp.concatenate(
        [jnp.broadcast_to(we.reshape(1, STACK, 1), (BI, STACK, 1)),
         jnp.broadcast_to(wet.reshape(1, STACK, 1), (BI, STACK, 1))],
        axis=2)
    h = jnp.concatenate(
        [e_blk[:, None, :], et_blk[:, None, :]], axis=1)
    z = jax.lax.dot_general(g, h, (((2,), (1,)), ((0,), (0,))),
                            preferred_element_type=F32)
    t = jnp.maximum(z + u[:, :, None] + vt[None, :, :], 0.0)
    t = t.astype(jnp.bfloat16)

    heads = [wai_ref[...], wao_ref[...]]
    if emit_edge:
        heads.append(we_out_ref[...])
    w3 = jnp.concatenate(heads, axis=1).T.astype(jnp.bfloat16)
    k = len(heads)
    w3b = jnp.broadcast_to(w3[None, :, :], (BI, k, STACK))
    proj = jax.lax.dot_general(w3b, t, (((2,), (1,)), ((0,), (0,))),
                               preferred_element_type=F32)
    att_i = jax.nn.sigmoid(proj[:, 0, :] + bvec[0])
    att_j = jax.nn.sigmoid(proj[:, 1, :] + bvec[1])
    mask = (a_ref[rows, :] != 0.0).astype(F32)
    if emit_edge:
        e1_scr[rows, :] = proj[:, 2, :] + bvec[2]
    w_in = mask * att_i
    w_out = mask * att_j
    min_scr[rows, :] = jax.lax.dot_general(
        t, w_in.astype(jnp.bfloat16), (((2,), (1,)), ((0,), (0,))),
        preferred_element_type=F32)
    mout_scr[...] += jnp.sum((t * w_out[:, None, :].astype(jnp.bfloat16)
                              ).astype(F32), axis=0)


def _node_update(x, min_scr, mout_scr, f, wn_ref, bn_ref):
    xn = jnp.dot(x, wn_ref[0:f, :], preferred_element_type=F32)
    xn += jnp.dot(min_scr[...], wn_ref[f:f + STACK, :],
                  preferred_element_type=F32)
    xn += jax.lax.dot_general(mout_scr[...], wn_ref[f + STACK:, :],
                              (((0,), (0,)), ((), ())),
                              preferred_element_type=F32)
    return xn + bn_ref[...]


def _make_fused(f1, node):
    def _fused_kernel(x_ref, a_ref, e_ref,
                      ws1_ref, bs1_ref, wai1_ref, bai1_ref, wao1_ref,
                      bao1_ref, wn1_ref, bn1_ref, we1_ref, be1_ref,
                      ws2_ref, bs2_ref, wai2_ref, bai2_ref, wao2_ref,
                      bao2_ref, wn2_ref, bn2_ref, we2_ref,
                      wd_ref, bd_ref, out_ref,
                      e1_scr, min_scr, mout_scr, x1_scr):
        s = pl.program_id(0)

        @pl.when(s == 0)
        def _zero():
            mout_scr[...] = jnp.zeros_like(mout_scr)

        @pl.when(s < 2)
        def _layer1():
            for half in range(2):
                _edge_step(s * 2 + half, x_ref, e_ref, a_ref, f1, ws1_ref,
                           bs1_ref, wai1_ref, wao1_ref, we1_ref,
                           (bai1_ref[0, 0], bao1_ref[0, 0], be1_ref[0, 0]),
                           e1_scr, min_scr, mout_scr, emit_edge=True)

        @pl.when(s == 2)
        def _node1():
            x1_scr[...] = _node_update(x_ref[...], min_scr, mout_scr, f1,
                                       wn1_ref, bn1_ref)
            mout_scr[...] = jnp.zeros_like(mout_scr)

        @pl.when(jnp.logical_and(s >= 3, s < 5))
        def _layer2():
            for half in range(2):
                _edge_step((s - 3) * 2 + half, x1_scr, e1_scr, a_ref, node,
                           ws2_ref, bs2_ref, wai2_ref, wao2_ref, we2_ref,
                           (bai2_ref[0, 0], bao2_ref[0, 0], 0.0),
                           None, min_scr, mout_scr, emit_edge=False)

        @pl.when(s == 5)
        def _node2():
            x2 = _node_update(x1_scr[...], min_scr, mout_scr, node,
                              wn2_ref, bn2_ref)
            out_ref[...] = jnp.dot(x2, wd_ref[...],
                                   preferred_element_type=F32) + bd_ref[...]

    return _fused_kernel


def kernel(x, a, e, Ws1, bs1, Wai1, bai1, Wao1, bao1, Wn1, bn1, We1, be1,
           Ws2, bs2, Wai2, bai2, Wao2, bao2, Wn2, bn2, We2, be2, Wd, bd):
    f1 = x.shape[2]
    node = Wn1.shape[1]
    nlab = Wd.shape[1]

    operands = [
        x[0], a[0], e[0, :, :, 0],
        Ws1, bs1.reshape(1, STACK), Wai1, bai1.reshape(1, 1), Wao1,
        bao1.reshape(1, 1), Wn1, bn1.reshape(1, node), We1, be1.reshape(1, 1),
        Ws2, bs2.reshape(1, STACK), Wai2, bai2.reshape(1, 1), Wao2,
        bao2.reshape(1, 1), Wn2, bn2.reshape(1, node), We2,
        Wd, bd.reshape(1, nlab),
    ]
    full = lambda s: pl.BlockSpec(s, lambda i: (0,) * len(s))
    out = pl.pallas_call(
        _make_fused(f1, node),
        grid=(6,),
        in_specs=[full(op.shape) for op in operands],
        out_shape=jax.ShapeDtypeStruct((N, nlab), F32),
        out_specs=full((N, nlab)),
        scratch_shapes=[
            pltpu.VMEM((N, N), F32),
            pltpu.VMEM((N, STACK), F32),
            pltpu.VMEM((STACK, N), F32),
            pltpu.VMEM((N, node), F32),
        ],
    )(*operands)
    return out[None, :, :]
